# baseline (speedup 1.0000x reference)
import functools

import jax
import jax.numpy as jnp
from jax import lax
from jax.experimental import pallas as pl
from jax.experimental.pallas import tpu as pltpu
from jax.experimental.pallas import tpu_sc as plsc

_VOCAB = 100000
_DIM = 128
_FUSE_ROWS = 400

_NC = 2
_NS = 16
_NW = _NC * _NS
_CHUNK = 128
_NBUF = 2
def _dyn_gather(v, idx):
    return lax.gather(
        v, idx[:, None],
        lax.GatherDimensionNumbers(
            offset_dims=(), collapsed_slice_dims=(0,), start_index_map=(0,)),
        (1,),
        mode=lax.GatherScatterMode.PROMISE_IN_BOUNDS,
    )


def _fuse_body(r_ref, i_ref, se0_ref, so0_ref, se1_ref, so1_ref, o_ref):
    r = r_ref[...]
    i = i_ref[...]
    dot = functools.partial(
        jax.lax.dot,
        precision=jax.lax.Precision.HIGHEST,
        preferred_element_type=jnp.float32,
    )
    o_ref[0] = dot(r, se0_ref[...]) + dot(i, so0_ref[...])
    o_ref[1] = dot(r, se1_ref[...]) + dot(i, so1_ref[...])


def _fuse_tables(embed_real, embed_imag):
    d = jnp.arange(_DIM)[:, None]
    c = jnp.arange(_DIM)[None, :]
    se0 = ((c == 2 * d) & (d < 64)).astype(jnp.float32)
    so0 = ((c == 2 * d + 1) & (d < 64)).astype(jnp.float32)
    se1 = ((c == 2 * (d - 64)) & (d >= 64)).astype(jnp.float32)
    so1 = ((c == 2 * (d - 64) + 1) & (d >= 64)).astype(jnp.float32)

    sel_spec = pl.BlockSpec((_DIM, _DIM), lambda g: (0, 0))
    return pl.pallas_call(
        _fuse_body,
        grid=(_VOCAB // _FUSE_ROWS,),
        in_specs=[
            pl.BlockSpec((_FUSE_ROWS, _DIM), lambda g: (g, 0)),
            pl.BlockSpec((_FUSE_ROWS, _DIM), lambda g: (g, 0)),
            sel_spec, sel_spec, sel_spec, sel_spec,
        ],
        out_specs=pl.BlockSpec((2, _FUSE_ROWS, _DIM), lambda g: (0, g, 0)),
        out_shape=jax.ShapeDtypeStruct((2, _VOCAB, _DIM), jnp.float32),
    )(embed_real, embed_imag, se0, so0, se1, so1)


def _gather_rows(table, ids_flat):
    n = ids_flat.shape[0]
    per_w = n // _NW
    n_chunks = per_w // _CHUNK
    n_groups = n_chunks // _NBUF
    mesh = plsc.VectorSubcoreMesh(core_axis_name="c", subcore_axis_name="s")

    @functools.partial(
        pl.kernel,
        out_type=jax.ShapeDtypeStruct((2 * n, _DIM), jnp.float32),
        mesh=mesh,
        scratch_types=[
            [pltpu.VMEM((_CHUNK,), jnp.int32) for _ in range(_NBUF)],
            [[pltpu.VMEM((_CHUNK,), jnp.int32) for _ in range(2)]
             for _ in range(_NBUF)],
            pltpu.VMEM((_NBUF, 2 * _CHUNK, _DIM), jnp.float32),
            pltpu.SemaphoreType.DMA,
            pltpu.SemaphoreType.DMA,
        ],
    )
    def k(table_hbm, ids_hbm, out_hbm, idv, idx2, rows_v, sem_g, sem_w):
        wid = lax.axis_index("s") * _NC + lax.axis_index("c")
        base = wid * per_w
        iota16 = lax.iota(jnp.int32, 16)
        dup_lo = iota16 >> 1
        dup_hi = (iota16 >> 1) + 8
        addv = (iota16 & 1) * _VOCAB

        def group(g, carry):
            gathers = []
            for b in range(_NBUF):
                start = base + (g * _NBUF + b) * _CHUNK

                @pl.when(g > 0)
                def _wait_prev_write():
                    pltpu.make_async_copy(
                        rows_v.at[b], out_hbm.at[pl.ds(0, 2 * _CHUNK)], sem_w
                    ).wait()

                pltpu.sync_copy(ids_hbm.at[pl.ds(start, _CHUNK)], idv[b])
                for s in range(_CHUNK // 16):
                    v = idv[b][pl.ds(16 * s, 16)]
                    g1 = _dyn_gather(v, dup_lo)
                    g2 = _dyn_gather(v, dup_hi)
                    half = (32 * s) // _CHUNK
                    col0 = (32 * s) % _CHUNK
                    idx2[b][half][pl.ds(col0, 16)] = g1 + addv
                    idx2[b][half][pl.ds(col0 + 16, 16)] = g2 + addv
                for h in range(2):
                    gathers.append(
                        pltpu.async_copy(
                            table_hbm.at[idx2[b][h]],
                            rows_v.at[b, pl.ds(h * _CHUNK, _CHUNK)],
                            sem_g,
                        )
                    )
            for b in range(_NBUF):
                start = base + (g * _NBUF + b) * _CHUNK
                gathers[2 * b].wait()
                gathers[2 * b + 1].wait()
                pltpu.async_copy(
                    rows_v.at[b], out_hbm.at[pl.ds(2 * start, 2 * _CHUNK)], sem_w
                )
            return carry

        lax.fori_loop(0, n_groups, group, 0, unroll=False)
        for b in range(_NBUF):
            pltpu.make_async_copy(
                rows_v.at[b], out_hbm.at[pl.ds(0, 2 * _CHUNK)], sem_w
            ).wait()

    return k(table, ids_flat)


def kernel(ids, embed_real, embed_imag):
    b, l = ids.shape
    fused = _fuse_tables(embed_real, embed_imag)
    table = fused.reshape(2 * _VOCAB, _DIM)
    ids_flat = ids.reshape(-1).astype(jnp.int32)
    rows = _gather_rows(table, ids_flat)
    return rows.reshape(b, l, _DIM, 2)


# speedup vs baseline: 120.3993x; 120.3993x over previous
import functools

import jax
import jax.numpy as jnp
from jax import lax
from jax.experimental import pallas as pl
from jax.experimental.pallas import tpu as pltpu
from jax.experimental.pallas import tpu_sc as plsc

_VOCAB = 100000
_DIM = 128

_NC = 2
_NS = 16
_NW = _NC * _NS
_CHUNK = 128
_NBUF = 2


def _dyn_gather(v, idx):
    return lax.gather(
        v, idx[:, None],
        lax.GatherDimensionNumbers(
            offset_dims=(), collapsed_slice_dims=(0,), start_index_map=(0,)),
        (1,),
        mode=lax.GatherScatterMode.PROMISE_IN_BOUNDS,
    )


def _gather_rows(table, ids_flat):
    n = ids_flat.shape[0]
    per_w = n // _NW
    n_chunks = per_w // _CHUNK
    n_groups = n_chunks // _NBUF
    mesh = plsc.VectorSubcoreMesh(core_axis_name="c", subcore_axis_name="s")

    @functools.partial(
        pl.kernel,
        out_type=jax.ShapeDtypeStruct((2 * n, _DIM), jnp.float32),
        mesh=mesh,
        scratch_types=[
            [pltpu.VMEM((_CHUNK,), jnp.int32) for _ in range(_NBUF)],
            [[pltpu.VMEM((_CHUNK,), jnp.int32) for _ in range(2)]
             for _ in range(_NBUF)],
            pltpu.VMEM((_NBUF, 2 * _CHUNK, _DIM), jnp.float32),
            pltpu.SemaphoreType.DMA,
            pltpu.SemaphoreType.DMA,
        ],
    )
    def k(table_hbm, ids_hbm, out_hbm, idv, idx2, rows_v, sem_g, sem_w):
        wid = lax.axis_index("s") * _NC + lax.axis_index("c")
        base = wid * per_w
        iota16 = lax.iota(jnp.int32, 16)
        dup_lo = iota16 >> 1
        dup_hi = (iota16 >> 1) + 8
        addv = (iota16 & 1) * _VOCAB

        def group(g, carry):
            gathers = []
            for b in range(_NBUF):
                start = base + (g * _NBUF + b) * _CHUNK

                @pl.when(g > 0)
                def _wait_prev_write():
                    pltpu.make_async_copy(
                        rows_v.at[b], out_hbm.at[pl.ds(0, 2 * _CHUNK)], sem_w
                    ).wait()

                pltpu.sync_copy(ids_hbm.at[pl.ds(start, _CHUNK)], idv[b])
                for s in range(_CHUNK // 16):
                    v = idv[b][pl.ds(16 * s, 16)]
                    g1 = _dyn_gather(v, dup_lo)
                    g2 = _dyn_gather(v, dup_hi)
                    half = (32 * s) // _CHUNK
                    col0 = (32 * s) % _CHUNK
                    idx2[b][half][pl.ds(col0, 16)] = g1 + addv
                    idx2[b][half][pl.ds(col0 + 16, 16)] = g2 + addv
                for h in range(2):
                    gathers.append(
                        pltpu.async_copy(
                            table_hbm.at[idx2[b][h]],
                            rows_v.at[b, pl.ds(h * _CHUNK, _CHUNK)],
                            sem_g,
                        )
                    )
            for b in range(_NBUF):
                start = base + (g * _NBUF + b) * _CHUNK
                gathers[2 * b].wait()
                gathers[2 * b + 1].wait()
                pltpu.async_copy(
                    rows_v.at[b], out_hbm.at[pl.ds(2 * start, 2 * _CHUNK)], sem_w
                )
            return carry

        lax.fori_loop(0, n_groups, group, 0, unroll=False)
        for b in range(_NBUF):
            pltpu.make_async_copy(
                rows_v.at[b], out_hbm.at[pl.ds(0, 2 * _CHUNK)], sem_w
            ).wait()

    return k(table, ids_flat)


def kernel(ids, embed_real, embed_imag):
    b, l = ids.shape
    table = jnp.concatenate([embed_real, embed_imag], axis=0)
    ids_flat = ids.reshape(-1).astype(jnp.int32)
    rows = _gather_rows(table, ids_flat)
    return rows.reshape(b, l, 2, _DIM).transpose(0, 1, 3, 2)
